# baseline (speedup 1.0000x reference)
import functools

import jax
import jax.numpy as jnp
from jax import lax
from jax.experimental import pallas as pl
from jax.experimental.pallas import tpu as pltpu
from jax.experimental.pallas import tpu_sc as plsc

B, S, D = 1024, 200, 64
N = B * S
VOCAB, HBK = 1000000, 10000
NC, NS, L = 2, 16, 16
NW = NC * NS
RPT = N // NW
C = 400
NCHUNK = RPT // C
G = C // L
EPS = 1e-5
MIX = 0.7


def _rsqrt(x):
    i = lax.bitcast_convert_type(x, jnp.int32)
    y = lax.bitcast_convert_type(
        jnp.int32(0x5F3759DF) - lax.shift_right_logical(i, 1), jnp.float32)
    for _ in range(3):
        y = y * (1.5 - 0.5 * x * y * y)
    return y


@functools.partial(
    pl.kernel,
    out_type=(
        jax.ShapeDtypeStruct((N, D), jnp.float32),
        jax.ShapeDtypeStruct((NW * 2 * L,), jnp.float32),
    ),
    mesh=plsc.VectorSubcoreMesh(core_axis_name="c", subcore_axis_name="s",
                                num_cores=NC, num_subcores=NS),
    compiler_params=pltpu.CompilerParams(use_tc_tiling_on_sc=False,
                                         needs_layout_passes=False),
    scratch_types=[
        pltpu.VMEM((2, C), jnp.int32),
        pltpu.VMEM((2, C), jnp.int32),
        pltpu.VMEM((2, C, D), jnp.float32),
        pltpu.VMEM((2, C, D), jnp.float32),
        pltpu.VMEM((3 * D + L,), jnp.float32),
        pltpu.VMEM((2 * L,), jnp.float32),
        pltpu.SemaphoreType.DMA,
        pltpu.SemaphoreType.DMA,
        pltpu.SemaphoreType.DMA,
        pltpu.SemaphoreType.DMA,
        pltpu.SemaphoreType.DMA,
        pltpu.SemaphoreType.DMA,
    ],
)
def _sc_forward(ids_hbm, tok_hbm, hsh_hbm, cons_hbm, out_hbm, part_hbm,
                tidx, hidx, buf, outb, cons, ptb,
                sem_t0, sem_t1, sem_a0, sem_a1, sem_o0, sem_o1):
    sem_t = (sem_t0, sem_t1)
    sem_a = (sem_a0, sem_a1)
    sem_o = (sem_o0, sem_o1)
    wid = lax.axis_index("s") * NC + lax.axis_index("c")
    base = wid * RPT
    pltpu.sync_copy(cons_hbm, cons)

    def load_ids(k, b):
        pltpu.sync_copy(ids_hbm.at[pl.ds(base + k * C, C)], tidx.at[b])

        def hx(i, acc):
            v = tidx[b, pl.ds(i * L, L)]
            hidx[b, pl.ds(i * L, L)] = (v * 31) % HBK
            return acc

        lax.fori_loop(0, G, hx, 0)

    def start_tok(k, b):
        return pltpu.async_copy(tok_hbm.at[tidx.at[b]], buf.at[b], sem_t[b])

    def start_hash_add(b):
        return pltpu.async_copy(hsh_hbm.at[hidx.at[b]], buf.at[b], sem_a[b],
                                add=True)

    load_ids(0, 0)
    start_tok(0, 0)
    load_ids(1, 1)
    pltpu.make_async_copy(tok_hbm.at[tidx.at[0]], buf.at[0], sem_t[0]).wait()
    start_hash_add(0)
    start_tok(1, 1)

    def compute_chunk(k, b, carry):
        bref = buf.at[b]
        oref = outb.at[b]
        NV = D // L
        w = [cons[pl.ds(j * L, L)] for j in range(NV)]
        gam = [cons[pl.ds(D + j * L, L)] for j in range(NV)]
        bet = [cons[pl.ds(2 * D + j * L, L)] for j in range(NV)]
        bgv = cons[pl.ds(3 * D, L)]
        UR = 8

        def row_blk(blk, acc):
            cnt, nrm = acc
            for u in range(UR):
                r = blk * UR + u
                c = [bref[r, pl.ds(j * L, L)] for j in range(NV)]
                lgv = ((c[0] * w[0] + bgv) + c[1] * w[1]) + (
                    c[2] * w[2] + c[3] * w[3])
                s1v = (c[0] + c[1]) + (c[2] + c[3])
                s2v = (c[0] * c[0] + c[1] * c[1]) + (
                    c[2] * c[2] + c[3] * c[3])
                lg = jnp.sum(lgv)
                s1 = jnp.sum(s1v)
                s2 = jnp.sum(s2v)
                m = jnp.where(lg > 0.0, 1.0, 0.0).astype(jnp.float32)
                mu = m * s1 * (MIX / D)
                var = m * s2 * (MIX * MIX / D) - mu * mu
                rinv = _rsqrt(var + EPS)
                av = jnp.full((L,), MIX * m * rinv, jnp.float32)
                bv2 = jnp.full((L,), mu * rinv, jnp.float32)
                o = [(av * c[j] - bv2) * gam[j] + bet[j] for j in range(NV)]
                qv = (o[0] * o[0] + o[1] * o[1]) + (o[2] * o[2] + o[3] * o[3])
                for j in range(NV):
                    oref[r, pl.ds(j * L, L)] = o[j]
                q = jnp.sum(qv)
                nq = jnp.where(q > 0.0, q * _rsqrt(q), 0.0)
                cnt = cnt + m
                nrm = nrm + nq
            return (cnt, nrm)

        return lax.fori_loop(0, C // UR, row_blk, carry)

    def loop_body(kk, carry):
        for b in (0, 1):
            k = 2 * kk + b
            @pl.when(k + 1 < NCHUNK)
            def _():
                pltpu.make_async_copy(tok_hbm.at[tidx.at[1 - b]],
                                      buf.at[1 - b], sem_t[1 - b]).wait()
                start_hash_add(1 - b)

            pltpu.make_async_copy(hsh_hbm.at[hidx.at[b]], buf.at[b],
                                  sem_a[b]).wait()

            @pl.when(kk >= 1)
            def _():
                pltpu.make_async_copy(
                    outb.at[b], out_hbm.at[pl.ds(base + (k - 2) * C, C)],
                    sem_o[b]).wait()

            carry = compute_chunk(k, b, carry)
            pltpu.async_copy(outb.at[b],
                             out_hbm.at[pl.ds(base + k * C, C)], sem_o[b])

            @pl.when(k + 2 < NCHUNK)
            def _():
                load_ids(k + 2, b)
                start_tok(k + 2, b)

        return carry

    cnt, nrm = lax.fori_loop(0, NCHUNK // 2, loop_body,
                             (jnp.float32(0.0), jnp.float32(0.0)))

    pltpu.make_async_copy(outb.at[0],
                          out_hbm.at[pl.ds(base + (NCHUNK - 2) * C, C)],
                          sem_o[0]).wait()
    pltpu.make_async_copy(outb.at[1],
                          out_hbm.at[pl.ds(base + (NCHUNK - 1) * C, C)],
                          sem_o[1]).wait()

    ptb[pl.ds(0, L)] = jnp.full((L,), cnt, jnp.float32)
    ptb[pl.ds(L, L)] = jnp.full((L,), nrm, jnp.float32)
    pltpu.sync_copy(ptb, part_hbm.at[pl.ds(wid * 2 * L, 2 * L)])


def kernel(input_ids, token_table, hash_table, Wg, bg, gamma, beta):
    ids = input_ids.reshape(N)
    hsh = hash_table * jnp.float32(0.3 / MIX)
    cons = jnp.concatenate([
        Wg.reshape(D).astype(jnp.float32) * jnp.float32(MIX),
        gamma.reshape(D).astype(jnp.float32),
        beta.reshape(D).astype(jnp.float32),
        bg.astype(jnp.float32).reshape(1),
        jnp.zeros((L - 1,), jnp.float32),
    ])
    out_flat, part = _sc_forward(ids, token_table, hsh, cons)
    out = out_flat.reshape(B, S, D)
    pm = part.reshape(NW, 2 * L)
    cnt = pm[:, :L].sum() / L
    nrm = pm[:, L:].sum() / L
    nf = jnp.float32(N)
    sparsity = (cnt / nf).astype(jnp.float32)
    plo = jnp.float32(1e-8)
    e0 = -(plo * jnp.log(plo) + (1 - plo) * jnp.log(1 - plo))
    gate_entropy = jnp.where(cnt > 0, jnp.float32(jnp.nan),
                             e0).astype(jnp.float32)
    emb_norm = (nrm / nf).astype(jnp.float32)
    return (out, sparsity, gate_entropy, emb_norm)


# speedup vs baseline: 1.2054x; 1.2054x over previous
import functools

import jax
import jax.numpy as jnp
from jax import lax
from jax.experimental import pallas as pl
from jax.experimental.pallas import tpu as pltpu
from jax.experimental.pallas import tpu_sc as plsc

B, S, D = 1024, 200, 64
N = B * S
VOCAB, HBK = 1000000, 10000
NC, NS, L = 2, 16, 16
NW = NC * NS
RPT = N // NW
C = 400
NCHUNK = RPT // C
G = C // L
EPS = 1e-5
MIX = 0.7


def _rsqrt(x):
    i = lax.bitcast_convert_type(x, jnp.int32)
    y = lax.bitcast_convert_type(
        jnp.int32(0x5F3759DF) - lax.shift_right_logical(i, 1), jnp.float32)
    for _ in range(3):
        y = y * (1.5 - 0.5 * x * y * y)
    return y


@functools.partial(
    pl.kernel,
    out_type=(
        jax.ShapeDtypeStruct((N, D), jnp.float32),
        jax.ShapeDtypeStruct((NW * 2 * L,), jnp.float32),
    ),
    mesh=plsc.VectorSubcoreMesh(core_axis_name="c", subcore_axis_name="s",
                                num_cores=NC, num_subcores=NS),
    compiler_params=pltpu.CompilerParams(use_tc_tiling_on_sc=False,
                                         needs_layout_passes=False),
    scratch_types=[
        pltpu.VMEM((2, C), jnp.int32),
        pltpu.VMEM((2, C), jnp.int32),
        pltpu.VMEM((2, C, D), jnp.float32),
        pltpu.VMEM((2, C, D), jnp.float32),
        pltpu.VMEM((3 * D + L,), jnp.float32),
        pltpu.VMEM((2 * L,), jnp.float32),
        pltpu.SemaphoreType.DMA,
        pltpu.SemaphoreType.DMA,
        pltpu.SemaphoreType.DMA,
        pltpu.SemaphoreType.DMA,
        pltpu.SemaphoreType.DMA,
        pltpu.SemaphoreType.DMA,
    ],
)
def _sc_forward(ids_hbm, tok_hbm, hsh_hbm, cons_hbm, out_hbm, part_hbm,
                tidx, hidx, buf, outb, cons, ptb,
                sem_t0, sem_t1, sem_a0, sem_a1, sem_o0, sem_o1):
    sem_t = (sem_t0, sem_t1)
    sem_a = (sem_a0, sem_a1)
    sem_o = (sem_o0, sem_o1)
    wid = lax.axis_index("s") * NC + lax.axis_index("c")
    base = wid * RPT
    pltpu.sync_copy(cons_hbm, cons)

    def load_ids(k, b):
        pltpu.sync_copy(ids_hbm.at[pl.ds(base + k * C, C)], tidx.at[b])

        def hx(i, acc):
            v = tidx[b, pl.ds(i * L, L)]
            hidx[b, pl.ds(i * L, L)] = (v * 31) % HBK
            return acc

        lax.fori_loop(0, G, hx, 0)

    def start_tok(k, b):
        return pltpu.async_copy(tok_hbm.at[tidx.at[b]], buf.at[b], sem_t[b])

    def start_hash_add(b):
        return pltpu.async_copy(hsh_hbm.at[hidx.at[b]], buf.at[b], sem_a[b],
                                add=True)

    load_ids(0, 0)
    start_tok(0, 0)
    load_ids(1, 1)
    pltpu.make_async_copy(tok_hbm.at[tidx.at[0]], buf.at[0], sem_t[0]).wait()
    start_hash_add(0)
    start_tok(1, 1)

    def compute_chunk(k, b, carry):
        bref = buf.at[b]
        oref = outb.at[b]
        NV = D // L
        w = [cons[pl.ds(j * L, L)] for j in range(NV)]
        gam = [cons[pl.ds(D + j * L, L)] for j in range(NV)]
        bet = [cons[pl.ds(2 * D + j * L, L)] for j in range(NV)]
        bgv = cons[pl.ds(3 * D, L)]
        UR = 4

        def row_blk(blk, acc):
            cnt, nrm = acc
            for u in range(UR):
                r = blk * UR + u
                c = [bref[r, pl.ds(j * L, L)] for j in range(NV)]
                lgv = ((c[0] * w[0] + bgv) + c[1] * w[1]) + (
                    c[2] * w[2] + c[3] * w[3])
                s1v = (c[0] + c[1]) + (c[2] + c[3])
                s2v = (c[0] * c[0] + c[1] * c[1]) + (
                    c[2] * c[2] + c[3] * c[3])
                lg = jnp.sum(lgv)
                s1 = jnp.sum(s1v)
                s2 = jnp.sum(s2v)
                m = jnp.where(lg > 0.0, 1.0, 0.0).astype(jnp.float32)
                mu = m * s1 * (MIX / D)
                var = m * s2 * (MIX * MIX / D) - mu * mu
                rinv = _rsqrt(var + EPS)
                av = jnp.full((L,), MIX * m * rinv, jnp.float32)
                bv2 = jnp.full((L,), mu * rinv, jnp.float32)
                o = [(av * c[j] - bv2) * gam[j] + bet[j] for j in range(NV)]
                qv = (o[0] * o[0] + o[1] * o[1]) + (o[2] * o[2] + o[3] * o[3])
                for j in range(NV):
                    oref[r, pl.ds(j * L, L)] = o[j]
                q = jnp.sum(qv)
                nq = jnp.where(q > 0.0, q * _rsqrt(q), 0.0)
                cnt = cnt + m
                nrm = nrm + nq
            return (cnt, nrm)

        return lax.fori_loop(0, C // UR, row_blk, carry)

    def loop_body(kk, carry):
        for b in (0, 1):
            k = 2 * kk + b
            @pl.when(k + 1 < NCHUNK)
            def _():
                pltpu.make_async_copy(tok_hbm.at[tidx.at[1 - b]],
                                      buf.at[1 - b], sem_t[1 - b]).wait()
                start_hash_add(1 - b)

            pltpu.make_async_copy(hsh_hbm.at[hidx.at[b]], buf.at[b],
                                  sem_a[b]).wait()

            @pl.when(kk >= 1)
            def _():
                pltpu.make_async_copy(
                    outb.at[b], out_hbm.at[pl.ds(base + (k - 2) * C, C)],
                    sem_o[b]).wait()

            carry = compute_chunk(k, b, carry)
            pltpu.async_copy(outb.at[b],
                             out_hbm.at[pl.ds(base + k * C, C)], sem_o[b])

            @pl.when(k + 2 < NCHUNK)
            def _():
                load_ids(k + 2, b)
                start_tok(k + 2, b)

        return carry

    cnt, nrm = lax.fori_loop(0, NCHUNK // 2, loop_body,
                             (jnp.float32(0.0), jnp.float32(0.0)))

    pltpu.make_async_copy(outb.at[0],
                          out_hbm.at[pl.ds(base + (NCHUNK - 2) * C, C)],
                          sem_o[0]).wait()
    pltpu.make_async_copy(outb.at[1],
                          out_hbm.at[pl.ds(base + (NCHUNK - 1) * C, C)],
                          sem_o[1]).wait()

    ptb[pl.ds(0, L)] = jnp.full((L,), cnt, jnp.float32)
    ptb[pl.ds(L, L)] = jnp.full((L,), nrm, jnp.float32)
    pltpu.sync_copy(ptb, part_hbm.at[pl.ds(wid * 2 * L, 2 * L)])


def kernel(input_ids, token_table, hash_table, Wg, bg, gamma, beta):
    ids = input_ids.reshape(N)
    hsh = hash_table * jnp.float32(0.3 / MIX)
    cons = jnp.concatenate([
        Wg.reshape(D).astype(jnp.float32) * jnp.float32(MIX),
        gamma.reshape(D).astype(jnp.float32),
        beta.reshape(D).astype(jnp.float32),
        bg.astype(jnp.float32).reshape(1),
        jnp.zeros((L - 1,), jnp.float32),
    ])
    out_flat, part = _sc_forward(ids, token_table, hsh, cons)
    out = out_flat.reshape(B, S, D)
    pm = part.reshape(NW, 2 * L)
    cnt = pm[:, :L].sum() / L
    nrm = pm[:, L:].sum() / L
    nf = jnp.float32(N)
    sparsity = (cnt / nf).astype(jnp.float32)
    plo = jnp.float32(1e-8)
    e0 = -(plo * jnp.log(plo) + (1 - plo) * jnp.log(1 - plo))
    gate_entropy = jnp.where(cnt > 0, jnp.float32(jnp.nan),
                             e0).astype(jnp.float32)
    emb_norm = (nrm / nf).astype(jnp.float32)
    return (out, sparsity, gate_entropy, emb_norm)
